# baseline (speedup 1.0000x reference)
import functools
import jax
import jax.numpy as jnp
from jax import lax
from jax.experimental import pallas as pl
from jax.experimental.pallas import tpu as pltpu
from jax.experimental.pallas import tpu_sc as plsc

NC = 2
NS = 16
L = 16
CH = 128
RB = 256
NW = NC * NS


def _make_kernel(B, V, F):
    assert F == 32 and B % (NW * RB) == 0 and V % 4 == 0
    b_per_w = B // NW
    n_rb = b_per_w // RB
    n_ch = b_per_w // CH
    mesh = plsc.VectorSubcoreMesh(
        core_axis_name="c", subcore_axis_name="s", num_cores=NC, num_subcores=NS
    )

    @functools.partial(
        pl.kernel,
        out_type=jax.ShapeDtypeStruct((B,), jnp.float32),
        mesh=mesh,
        compiler_params=pltpu.CompilerParams(needs_layout_passes=False),
        scratch_types=[
            pltpu.VMEM((n_ch, CH), jnp.int32),
            pltpu.VMEM((n_ch, CH), jnp.int32),
            pltpu.VMEM((n_ch, CH), jnp.int32),
            pltpu.VMEM((n_ch, CH), jnp.int32),
            pltpu.VMEM((RB, 128), jnp.float32),
            pltpu.VMEM((RB, 128), jnp.float32),
            pltpu.VMEM((b_per_w,), jnp.float32),
            pltpu.SemaphoreType.DMA,
        ],
    )
    def k(n1_hbm, n2_hbm, tab_hbm, out_hbm,
          idx1_v, idx2_v, q1_v, q2_v, rows1_v, rows2_v, out_v, sem):
        wid = lax.axis_index("s") * NC + lax.axis_index("c")
        base = wid * b_per_w
        crow = wid * n_ch

        d1 = pltpu.async_copy(n1_hbm.at[pl.ds(crow, n_ch)], idx1_v, sem)
        d2 = pltpu.async_copy(n2_hbm.at[pl.ds(crow, n_ch)], idx2_v, sem)
        d1.wait()
        d2.wait()

        for j in range(n_ch):
            for c in range(CH // L):
                s = pl.ds(c * L, L)
                q1_v[j, s] = idx1_v[j, s] >> 2
                q2_v[j, s] = idx2_v[j, s] >> 2

        lane = lax.iota(jnp.int32, 16)

        for rb in range(n_rb):
            descs = []
            for j in range(RB // CH):
                jj = rb * (RB // CH) + j
                descs.append(
                    pltpu.async_copy(
                        tab_hbm.at[q1_v.at[jj]],
                        rows1_v.at[pl.ds(j * CH, CH)], sem,
                    )
                )
                descs.append(
                    pltpu.async_copy(
                        tab_hbm.at[q2_v.at[jj]],
                        rows2_v.at[pl.ds(j * CH, CH)], sem,
                    )
                )
            for d in descs:
                d.wait()

            for g in range(RB // L):
                r0 = rb * RB + g * L
                iv1 = idx1_v[r0 // CH, pl.ds(r0 % CH, L)]
                iv2 = idx2_v[r0 // CH, pl.ds(r0 % CH, L)]
                col1 = (iv1 & 3) << 5
                col2 = (iv2 & 3) << 5
                row = lane + g * L
                acc = jnp.zeros((L,), jnp.float32)
                for f in range(F):
                    a = plsc.load_gather(rows1_v, [row, col1 + f])
                    b = plsc.load_gather(rows2_v, [row, col2 + f])
                    acc = acc + a * b
                out_v[pl.ds(r0, L)] = acc

        pltpu.sync_copy(out_v, out_hbm.at[pl.ds(base, b_per_w)])

    return k


@jax.jit
def kernel(node1, node2, node_factors):
    B = node1.shape[0]
    V, F = node_factors.shape
    n1 = node1.reshape(B // CH, CH)
    n2 = node2.reshape(B // CH, CH)
    tab128 = node_factors.reshape(V // 4, 128)
    k = _make_kernel(B, V, F)
    return k(n1, n2, tab128)
